# baseline (speedup 1.0000x reference)
import functools

import jax
import jax.numpy as jnp
from jax import lax
from jax.experimental import pallas as pl
from jax.experimental.pallas import tpu as pltpu
from jax.experimental.pallas import tpu_sc as plsc

_B = 16384
_H = 32
_NW = 32
_BPW = _B // _NW
_CH = 64
_NCH = _BPW // _CH
_LMBDA = 0.0001


def _sc_body(h2, t2, r2, e1, e2, rl1, rl2,
             res_out, regul_out,
             hraw, traw, rraw, hdiv, tdiv, rdiv,
             be1h, be2h, be1t, be2t, br1, br2,
             resbuf, accbuf, sems):
    nc = 2
    wid = lax.axis_index("s") * nc + lax.axis_index("c")

    pltpu.sync_copy(h2.at[wid], hraw)
    pltpu.sync_copy(t2.at[wid], traw)
    pltpu.sync_copy(r2.at[wid], rraw)

    def div_body(i, carry):
        for raw, dv in ((hraw, hdiv), (traw, tdiv), (rraw, rdiv)):
            v = raw[pl.ds(i * 16, 16)]
            dv[i >> 2, pl.ds((i & 3) * 16, 16)] = jnp.right_shift(v, 2)
        return carry

    lax.fori_loop(0, _BPW // 16, div_body, 0)

    def fire(c, slot):
        sem = sems.at[slot]
        return [
            pltpu.async_copy(e1.at[hdiv.at[c]], be1h.at[slot], sem),
            pltpu.async_copy(e2.at[hdiv.at[c]], be2h.at[slot], sem),
            pltpu.async_copy(e1.at[tdiv.at[c]], be1t.at[slot], sem),
            pltpu.async_copy(e2.at[tdiv.at[c]], be2t.at[slot], sem),
            pltpu.async_copy(rl1.at[rdiv.at[c]], br1.at[slot], sem),
            pltpu.async_copy(rl2.at[rdiv.at[c]], br2.at[slot], sem),
        ]

    lanes = lax.iota(jnp.int32, 16)
    zero16 = jnp.zeros((16,), jnp.float32)

    def compute(c, slot, acc):
        base = c * _CH

        def grp(g, acc_g):
            rows = lanes + g * 16
            hm = (hraw[pl.ds(base + g * 16, 16)] & 3) * 32
            tm = (traw[pl.ds(base + g * 16, 16)] & 3) * 32
            rm = (rraw[pl.ds(base + g * 16, 16)] & 3) * 32

            def jbody(j, carry):
                res16, acc_j = carry
                ch = hm + j
                ct = tm + j
                cr = rm + j
                a = plsc.load_gather(be1h.at[slot], [rows, ch])
                b = plsc.load_gather(be2h.at[slot], [rows, ch])
                c_ = plsc.load_gather(be1t.at[slot], [rows, ct])
                d = plsc.load_gather(be2t.at[slot], [rows, ct])
                p = plsc.load_gather(br1.at[slot], [rows, cr])
                q = plsc.load_gather(br2.at[slot], [rows, cr])
                res16 = res16 + (a * c_ + b * d) * p + (a * d - b * c_) * q
                acc_j = acc_j + a * a + b * b + c_ * c_ + d * d + p * p + q * q
                return res16, acc_j

            res16, acc_g = lax.fori_loop(0, _H, jbody, (zero16, acc_g),
                                         unroll=2)
            resbuf[pl.ds(base + g * 16, 16)] = res16
            return acc_g

        return lax.fori_loop(0, _CH // 16, grp, acc)

    acc = zero16
    cps = fire(0, 0)
    for c in range(_NCH):
        nxt = fire(c + 1, (c + 1) % 2) if c + 1 < _NCH else []
        for cp in cps:
            cp.wait()
        acc = compute(c, c % 2, acc)
        cps = nxt

    accbuf[...] = acc
    pltpu.sync_copy(resbuf, res_out.at[wid])
    pltpu.sync_copy(accbuf, regul_out.at[wid])


@jax.jit
def _sc_call(h2, t2, r2, e1, e2, rl1, rl2):
    mesh = plsc.VectorSubcoreMesh(core_axis_name="c", subcore_axis_name="s")
    return pl.kernel(
        _sc_body,
        out_type=[
            jax.ShapeDtypeStruct((_NW, _BPW), jnp.float32),
            jax.ShapeDtypeStruct((_NW, 16), jnp.float32),
        ],
        mesh=mesh,
        compiler_params=pltpu.CompilerParams(needs_layout_passes=False),
        scratch_types=[
            pltpu.VMEM((_BPW,), jnp.int32),
            pltpu.VMEM((_BPW,), jnp.int32),
            pltpu.VMEM((_BPW,), jnp.int32),
            pltpu.VMEM((_NCH, _CH), jnp.int32),
            pltpu.VMEM((_NCH, _CH), jnp.int32),
            pltpu.VMEM((_NCH, _CH), jnp.int32),
            pltpu.VMEM((2, _CH, 128), jnp.float32),
            pltpu.VMEM((2, _CH, 128), jnp.float32),
            pltpu.VMEM((2, _CH, 128), jnp.float32),
            pltpu.VMEM((2, _CH, 128), jnp.float32),
            pltpu.VMEM((2, _CH, 128), jnp.float32),
            pltpu.VMEM((2, _CH, 128), jnp.float32),
            pltpu.VMEM((_BPW,), jnp.float32),
            pltpu.VMEM((16,), jnp.float32),
            pltpu.SemaphoreType.DMA((2,)),
        ],
    )(h2, t2, r2, e1, e2, rl1, rl2)


def _tc_body(res_ref, y_ref, part_ref, out_ref):
    x = -(y_ref[...] * res_ref[...])
    sp = jnp.maximum(x, 0.0) + jnp.log1p(jnp.exp(-jnp.abs(x)))
    lf = jnp.sum(sp) * (1.0 / _B)
    reg = jnp.sum(part_ref[...]) * (1.0 / (_B * _H))
    out_ref[...] = jnp.reshape(lf + _LMBDA * reg, (1, 1))


def kernel(h, t, r, y, ent1, ent2, rel1, rel2):
    h2 = h.reshape(_NW, _BPW)
    t2 = t.reshape(_NW, _BPW)
    r2 = r.reshape(_NW, _BPW)
    e1 = ent1.reshape(-1, 128)
    e2 = ent2.reshape(-1, 128)
    rl1 = rel1.reshape(-1, 128)
    rl2 = rel2.reshape(-1, 128)
    res, parts = _sc_call(h2, t2, r2, e1, e2, rl1, rl2)
    res2 = res.reshape(128, 128)
    y2 = y.reshape(128, 128)
    out = pl.pallas_call(
        _tc_body,
        out_shape=jax.ShapeDtypeStruct((1, 1), jnp.float32),
    )(res2, y2, parts)
    return out[0, 0]


# speedup vs baseline: 1.4278x; 1.4278x over previous
import jax
import jax.numpy as jnp
from jax import lax
from jax.experimental import pallas as pl
from jax.experimental.pallas import tpu as pltpu
from jax.experimental.pallas import tpu_sc as plsc

_B = 16384
_H = 32
_NW = 32
_BPW = _B // _NW
_CH = 64
_NCH = _BPW // _CH
_LMBDA = 0.0001


def _sc_body(h2, t2, r2, y_in, ent1, ent2, rel1, rel2,
             res_out, regul_out,
             hraw, traw, rraw,
             be1h, be2h, be1t, be2t, br1, br2,
             drainbuf, resbuf, accbuf, sems):
    nc = 2
    wid = lax.axis_index("s") * nc + lax.axis_index("c")

    pltpu.sync_copy(h2.at[wid], hraw)
    pltpu.sync_copy(t2.at[wid], traw)
    pltpu.sync_copy(r2.at[wid], rraw)

    def fire(c, slot):
        base = c * _CH
        sem = sems.at[slot]

        def grp16(g, carry):
            hv = hraw[pl.ds(base + g * 16, 16)]
            tv = traw[pl.ds(base + g * 16, 16)]
            rv = rraw[pl.ds(base + g * 16, 16)]
            for l in range(16):
                b = g * 16 + l
                ih = hv[l]
                it = tv[l]
                ir = rv[l]
                pltpu.async_copy(
                    ent1.at[ih], be1h.at[slot, b, pl.ds(0, _H)], sem)
                pltpu.async_copy(
                    ent2.at[ih], be2h.at[slot, b, pl.ds(0, _H)], sem)
                pltpu.async_copy(
                    ent1.at[it], be1t.at[slot, b, pl.ds(0, _H)], sem)
                pltpu.async_copy(
                    ent2.at[it], be2t.at[slot, b, pl.ds(0, _H)], sem)
                pltpu.async_copy(
                    rel1.at[ir], br1.at[slot, b, pl.ds(0, _H)], sem)
                pltpu.async_copy(
                    rel2.at[ir], br2.at[slot, b, pl.ds(0, _H)], sem)
            return carry

        lax.fori_loop(0, _CH // 16, grp16, 0)

    def drain(slot):
        pltpu.make_async_copy(
            y_in.at[pl.ds(0, 6 * _CH * _H)],
            drainbuf,
            sems.at[slot],
        ).wait()

    lanes = lax.iota(jnp.int32, 16)
    zero16 = jnp.zeros((16,), jnp.float32)
    zero16i = jnp.zeros((16,), jnp.int32)

    def compute(c, slot, acc):
        base = c * _CH

        def grp(g, acc_g):
            rows = lanes + g * 16

            def jbody(j, carry):
                res16, acc_j = carry
                jv = zero16i + j
                a = plsc.load_gather(be1h.at[slot], [rows, jv])
                b = plsc.load_gather(be2h.at[slot], [rows, jv])
                c_ = plsc.load_gather(be1t.at[slot], [rows, jv])
                d = plsc.load_gather(be2t.at[slot], [rows, jv])
                p = plsc.load_gather(br1.at[slot], [rows, jv])
                q = plsc.load_gather(br2.at[slot], [rows, jv])
                res16 = res16 + (a * c_ + b * d) * p + (a * d - b * c_) * q
                acc_j = acc_j + a * a + b * b + c_ * c_ + d * d + p * p + q * q
                return res16, acc_j

            res16, acc_g = lax.fori_loop(0, _H, jbody, (zero16, acc_g),
                                         unroll=2)
            resbuf[pl.ds(base + g * 16, 16)] = res16
            return acc_g

        return lax.fori_loop(0, _CH // 16, grp, acc)

    acc = zero16
    fire(0, 0)
    for c in range(_NCH):
        if c + 1 < _NCH:
            fire(c + 1, (c + 1) % 2)
        drain(c % 2)
        acc = compute(c, c % 2, acc)

    accbuf[...] = acc
    pltpu.sync_copy(resbuf, res_out.at[wid])
    pltpu.sync_copy(accbuf, regul_out.at[wid])


@jax.jit
def _sc_call(h2, t2, r2, y_in, ent1, ent2, rel1, rel2):
    mesh = plsc.VectorSubcoreMesh(core_axis_name="c", subcore_axis_name="s")
    return pl.kernel(
        _sc_body,
        out_type=[
            jax.ShapeDtypeStruct((_NW, _BPW), jnp.float32),
            jax.ShapeDtypeStruct((_NW, 16), jnp.float32),
        ],
        mesh=mesh,
        compiler_params=pltpu.CompilerParams(needs_layout_passes=False),
        scratch_types=[
            pltpu.VMEM((_BPW,), jnp.int32),
            pltpu.VMEM((_BPW,), jnp.int32),
            pltpu.VMEM((_BPW,), jnp.int32),
            pltpu.VMEM((2, _CH, 128), jnp.float32),
            pltpu.VMEM((2, _CH, 128), jnp.float32),
            pltpu.VMEM((2, _CH, 128), jnp.float32),
            pltpu.VMEM((2, _CH, 128), jnp.float32),
            pltpu.VMEM((2, _CH, 128), jnp.float32),
            pltpu.VMEM((2, _CH, 128), jnp.float32),
            pltpu.VMEM((6 * _CH * _H,), jnp.float32),
            pltpu.VMEM((_BPW,), jnp.float32),
            pltpu.VMEM((16,), jnp.float32),
            pltpu.SemaphoreType.DMA((2,)),
        ],
    )(h2, t2, r2, y_in, ent1, ent2, rel1, rel2)


def _tc_body(res_ref, y_ref, part_ref, out_ref):
    x = -(y_ref[...] * res_ref[...])
    sp = jnp.maximum(x, 0.0) + jnp.log1p(jnp.exp(-jnp.abs(x)))
    lf = jnp.sum(sp) * (1.0 / _B)
    reg = jnp.sum(part_ref[...]) * (1.0 / (_B * _H))
    out_ref[...] = jnp.reshape(lf + _LMBDA * reg, (1, 1))


def kernel(h, t, r, y, ent1, ent2, rel1, rel2):
    h2 = h.reshape(_NW, _BPW)
    t2 = t.reshape(_NW, _BPW)
    r2 = r.reshape(_NW, _BPW)
    res, parts = _sc_call(h2, t2, r2, y, ent1, ent2, rel1, rel2)
    res2 = res.reshape(128, 128)
    y2 = y.reshape(128, 128)
    out = pl.pallas_call(
        _tc_body,
        out_shape=jax.ShapeDtypeStruct((1, 1), jnp.float32),
    )(res2, y2, parts)
    return out[0, 0]
